# baseline (speedup 1.0000x reference)
import functools

import jax
import jax.numpy as jnp
from jax import lax
from jax.experimental import pallas as pl
from jax.experimental.pallas import tpu as pltpu
from jax.experimental.pallas import tpu_sc as plsc

NC = 2
NS = 16
NW = NC * NS
C = 256


def _log_1to2(x):
    s = (x - 1.0) / (x + 1.0)
    s2 = s * s
    p = 1.0 / 9.0
    p = p * s2 + 1.0 / 7.0
    p = p * s2 + 1.0 / 5.0
    p = p * s2 + 1.0 / 3.0
    p = p * s2 + 1.0
    return (2.0 * s) * p


def _sqrt(x):
    i = lax.bitcast_convert_type(x, jnp.int32)
    y = lax.bitcast_convert_type(jnp.int32(0x5F3759DF) - (i >> 1), jnp.float32)
    for _ in range(3):
        y = y * (1.5 - 0.5 * x * y * y)
    return x * y


def kernel(pos_idxs, ys, neg_idxs, num_neg, W):
    B = pos_idxs.shape[1]
    NT = neg_idxs.shape[1]
    D = W.shape[1]
    ppw = B // NW
    npw = NT // NW
    assert ppw % C == 0 and npw % C == 0 and D % 16 == 0
    scale = 1.0 / float(B + NT)

    mesh = plsc.VectorSubcoreMesh(core_axis_name="c", subcore_axis_name="s")

    @functools.partial(
        pl.kernel,
        mesh=mesh,
        compiler_params=pltpu.CompilerParams(
            needs_layout_passes=False, use_tc_tiling_on_sc=False),
        out_type=jax.ShapeDtypeStruct((NW, 16), jnp.float32),
        scratch_types=[
            pltpu.VMEM((C,), jnp.int32),
            pltpu.VMEM((C,), jnp.int32),
            pltpu.VMEM((C, D), jnp.float32),
            pltpu.VMEM((C, D), jnp.float32),
            pltpu.VMEM((C,), jnp.float32),
            pltpu.VMEM((16,), jnp.float32),
            pltpu.SemaphoreType.DMA,
            pltpu.SemaphoreType.DMA,
        ],
    )
    def sc_loss(pos0_h, pos1_h, ys_h, neg0_h, neg1_h, w_h, out_h,
                idxu, idxv, urows, vrows, ysv, accv, sem0, sem1):
        wid = lax.axis_index("s") * NC + lax.axis_index("c")
        lanes = lax.iota(jnp.int32, 16)

        def chunk(i0_h, i1_h, base, is_pos, acc):
            pltpu.sync_copy(i0_h.at[pl.ds(base, C)], idxu)
            pltpu.sync_copy(i1_h.at[pl.ds(base, C)], idxv)
            if is_pos:
                pltpu.sync_copy(ys_h.at[pl.ds(base, C)], ysv)
            cu = pltpu.async_copy(w_h.at[idxu], urows, sem0)
            cv = pltpu.async_copy(w_h.at[idxv], vrows, sem1)
            cu.wait()
            cv.wait()

            def group(g, acc):
                rvec = g * 16 + lanes
                dot = jnp.zeros((16,), jnp.float32)
                for j in range(D):
                    cvec = jnp.full((16,), j, jnp.int32)
                    au = plsc.load_gather(urows, [rvec, cvec])
                    av = plsc.load_gather(vrows, [rvec, cvec])
                    dot = dot + au * av
                z = -dot if is_pos else dot
                t = jnp.exp(-jnp.abs(z))
                sp = jnp.maximum(z, 0.0) + _log_1to2(1.0 + t)
                if is_pos:
                    yv = plsc.load_gather(ysv, [rvec])
                    sp = (_log_1to2(1.0 + _sqrt(yv)) + 1.0) * sp
                return acc + sp

            return lax.fori_loop(0, C // 16, group, acc)

        acc = jnp.zeros((16,), jnp.float32)
        acc = lax.fori_loop(
            0, ppw // C,
            lambda c, a: chunk(pos0_h, pos1_h, wid * ppw + c * C, True, a),
            acc)
        acc = lax.fori_loop(
            0, npw // C,
            lambda c, a: chunk(neg0_h, neg1_h, wid * npw + c * C, False, a),
            acc)
        accv[...] = acc * scale
        pltpu.sync_copy(accv, out_h.at[wid])

    partials = sc_loss(pos_idxs[0], pos_idxs[1], ys, neg_idxs[0], neg_idxs[1], W)
    return jnp.sum(partials)
